# baseline (speedup 1.0000x reference)
import numpy as np
import jax
import jax.numpy as jnp
from jax.experimental import pallas as pl
from jax.experimental.pallas import tpu as pltpu

_PACK = 8
_CHAINS = 2


def _round_up(n, m):
    return ((n + m - 1) // m) * m


def _col_compact_sel(w, pack):
    S = np.zeros((w, w // 2), np.float32)
    S[2 * np.arange(w // 2), np.arange(w // 2)] = 1.0
    return np.kron(np.eye(pack, dtype=np.float32), S)


def _shift_lr(x, img_w):
    R, L = x.shape
    z = jnp.zeros((R, 1), x.dtype)
    col = jax.lax.broadcasted_iota(jnp.int32, (1, L), 1) % img_w
    not_first = jnp.where(col == 0, 0.0, 1.0).astype(x.dtype)
    not_last = jnp.where(col == img_w - 1, 0.0, 1.0).astype(x.dtype)
    xl = jnp.concatenate([x[:, 1:], z], axis=1) * not_last
    xr = jnp.concatenate([z, x[:, :-1]], axis=1) * not_first
    return xr, xl


def _pool_rows(y, scratches):
    R, L = y.shape
    nc = L // 128
    for c in range(nc):
        scratches[c][:R, :] = y[:, c * 128:(c + 1) * 128]
    ev = jnp.concatenate(
        [scratches[c][pl.ds(0, R // 2, 2), :] for c in range(nc)], axis=1)
    od = jnp.concatenate(
        [scratches[c][pl.ds(1, R // 2, 2), :] for c in range(nc)], axis=1)
    return jnp.maximum(ev, od)


def _col_neighbor_max(t):
    return jnp.maximum(t, jnp.concatenate([t[:, 1:], t[:, :1]], axis=1))


def _conv_chain(x, m1_ref, b1_ref, m2_ref, b2_ref, s1_ref, scratches):
    f32 = jnp.float32
    bf16 = jnp.bfloat16

    xr, xl = _shift_lr(x, 64)
    xs = jnp.concatenate([xr, x, xl], axis=0)

    y = jnp.dot(m1_ref[...], xs, preferred_element_type=f32)
    y = jnp.maximum(y + b1_ref[...], 0.0)

    tr = _pool_rows(y, scratches)
    u = _col_neighbor_max(tr).astype(bf16)
    p1 = jnp.dot(u, s1_ref[...], preferred_element_type=f32)
    p1 = p1.astype(bf16)

    p1r, p1l = _shift_lr(p1, 32)
    ps = jnp.concatenate([p1r, p1, p1l], axis=0)

    y2 = jnp.dot(m2_ref[...], ps, preferred_element_type=f32)
    y2 = jnp.maximum(y2 + b2_ref[...], 0.0)

    tr2 = _pool_rows(y2, scratches)
    return _col_neighbor_max(tr2).astype(bf16)


def _conv_stack_kernel(x_ref, m1_ref, b1_ref, m2_ref, b2_ref, s1_ref,
                       out_ref, *scratches):
    ns = len(scratches) // _CHAINS
    for u in range(_CHAINS):
        out_ref[0, u, :, :] = _conv_chain(
            x_ref[0, u, :, :], m1_ref, b1_ref, m2_ref, b2_ref, s1_ref,
            scratches[u * ns:(u + 1) * ns])


def _fc_stack_kernel(x_ref, w1_ref, b1_ref, w2_ref, b2_ref, w3_ref, b3_ref,
                     o_ref):
    f32 = jnp.float32
    h = jnp.dot(x_ref[...], w1_ref[...], preferred_element_type=f32)
    h = jnp.maximum(h + b1_ref[...], 0.0)
    h = jnp.dot(h, w2_ref[...], preferred_element_type=f32)
    h = jnp.maximum(h + b2_ref[...], 0.0)
    o = jnp.dot(h, w3_ref[...], preferred_element_type=f32) + b3_ref[...]
    o_ref[...] = o.astype(o_ref.dtype)


def kernel(x, m1_0, m1_1, m1_2, c1_0, c1_2, b1s, re1, ro1, pe1, po1,
           m2_0, m2_1, m2_2, c2_0, c2_2, b2s, re2, ro2, pe2, po2,
           fc1_w, fc1_b, fc2_w, fc2_b, fc3_w, fc3_b):
    f32 = jnp.float32
    bf16 = jnp.bfloat16

    N = x.shape[0]
    assert x.shape[1:] == (3, 64, 64), x.shape
    G = _PACK * _CHAINS
    Np = _round_up(N, G)
    x = x.astype(bf16)
    if Np != N:
        x = jnp.pad(x, ((0, Np - N), (0, 0), (0, 0), (0, 0)))
    Nb = Np // G

    xp = x.reshape(Nb, _CHAINS, _PACK, 3, 64, 64).transpose(0, 1, 3, 4, 2, 5)
    xp = xp.reshape(Nb, _CHAINS, 3 * 64, _PACK * 64)

    m1 = jnp.concatenate([m1_0, m1_1, m1_2], axis=1).astype(bf16)
    m2 = jnp.concatenate([m2_0, m2_1, m2_2], axis=1).astype(bf16)
    s1 = jnp.asarray(_col_compact_sel(64, _PACK), bf16)

    nsc = _PACK * 64 // 128
    conv_out = pl.pallas_call(
        _conv_stack_kernel,
        out_shape=jax.ShapeDtypeStruct((Nb, _CHAINS, 12 * 16, _PACK * 32),
                                       bf16),
        grid=(Nb,),
        in_specs=[
            pl.BlockSpec((1, _CHAINS, 3 * 64, _PACK * 64),
                         lambda i: (i, 0, 0, 0)),
            pl.BlockSpec(m1.shape, lambda i: (0, 0)),
            pl.BlockSpec(b1s.shape, lambda i: (0, 0)),
            pl.BlockSpec(m2.shape, lambda i: (0, 0)),
            pl.BlockSpec(b2s.shape, lambda i: (0, 0)),
            pl.BlockSpec(s1.shape, lambda i: (0, 0)),
        ],
        out_specs=pl.BlockSpec((1, _CHAINS, 12 * 16, _PACK * 32),
                               lambda i: (i, 0, 0, 0)),
        scratch_shapes=[pltpu.VMEM((6 * 64, 128), f32)
                        for _ in range(_CHAINS * nsc)],
        compiler_params=pltpu.CompilerParams(dimension_semantics=("parallel",)),
    )(xp, m1, b1s.astype(f32), m2, b2s.astype(f32), s1)

    feat = conv_out[..., ::2].reshape(Nb, _CHAINS, 12, 16, _PACK, 16)
    feat = feat.transpose(0, 1, 4, 2, 3, 5)
    flat = feat.reshape(Np, 12 * 16 * 16)[:N]

    K = flat.shape[1]
    n1 = fc1_w.shape[1]
    n2 = fc2_w.shape[1]
    n3 = fc3_w.shape[1]

    TB = min(128, _round_up(N, 8))
    Nf = _round_up(N, TB)
    if Nf != N:
        flat = jnp.pad(flat, ((0, Nf - N), (0, 0)))

    out = pl.pallas_call(
        _fc_stack_kernel,
        out_shape=jax.ShapeDtypeStruct((Nf, n3), f32),
        grid=(Nf // TB,),
        in_specs=[
            pl.BlockSpec((TB, K), lambda i: (i, 0)),
            pl.BlockSpec((K, n1), lambda i: (0, 0)),
            pl.BlockSpec((1, n1), lambda i: (0, 0)),
            pl.BlockSpec((n1, n2), lambda i: (0, 0)),
            pl.BlockSpec((1, n2), lambda i: (0, 0)),
            pl.BlockSpec((n2, n3), lambda i: (0, 0)),
            pl.BlockSpec((1, n3), lambda i: (0, 0)),
        ],
        out_specs=pl.BlockSpec((TB, n3), lambda i: (i, 0)),
        compiler_params=pltpu.CompilerParams(dimension_semantics=("parallel",)),
    )(flat, fc1_w.astype(bf16), fc1_b.astype(f32),
      fc2_w.astype(f32), fc2_b.astype(f32),
      fc3_w.astype(f32), fc3_b.astype(f32))
    return out[:N]


# speedup vs baseline: 1.3741x; 1.3741x over previous
import numpy as np
import jax
import jax.numpy as jnp
from jax.experimental import pallas as pl
from jax.experimental.pallas import tpu as pltpu

_PACK = 8
_CHAINS = 2


def _round_up(n, m):
    return ((n + m - 1) // m) * m


def _col_compact_sel(w, pack):
    S = np.zeros((w, w // 2), np.float32)
    S[2 * np.arange(w // 2), np.arange(w // 2)] = 1.0
    return np.kron(np.eye(pack, dtype=np.float32), S)


def _shift_lr(x, img_w):
    R, L = x.shape
    z = jnp.zeros((R, 1), x.dtype)
    col = jax.lax.broadcasted_iota(jnp.int32, (1, L), 1) % img_w
    not_first = jnp.where(col == 0, 0.0, 1.0).astype(x.dtype)
    not_last = jnp.where(col == img_w - 1, 0.0, 1.0).astype(x.dtype)
    xl = jnp.concatenate([x[:, 1:], z], axis=1) * not_last
    xr = jnp.concatenate([z, x[:, :-1]], axis=1) * not_first
    return xr, xl


def _pool_rows(y, scratches):
    R, L = y.shape
    nc = L // 128
    for c in range(nc):
        scratches[c][:R, :] = y[:, c * 128:(c + 1) * 128]
    ev = jnp.concatenate(
        [scratches[c][pl.ds(0, R // 2, 2), :] for c in range(nc)], axis=1)
    od = jnp.concatenate(
        [scratches[c][pl.ds(1, R // 2, 2), :] for c in range(nc)], axis=1)
    return jnp.maximum(ev, od)


def _col_neighbor_max(t):
    return jnp.maximum(t, jnp.concatenate([t[:, 1:], t[:, :1]], axis=1))


def _conv_chain(x, m1_ref, b1_ref, m2_ref, b2_ref, s1_ref, s2_ref, scratches):
    f32 = jnp.float32
    bf16 = jnp.bfloat16

    xr, xl = _shift_lr(x, 64)
    xs = jnp.concatenate([xr, x, xl], axis=0)

    y = jnp.dot(m1_ref[...], xs, preferred_element_type=f32)
    y = jnp.maximum(y + b1_ref[...], 0.0)

    tr = _pool_rows(y, scratches)
    u = _col_neighbor_max(tr).astype(bf16)
    p1 = jnp.dot(u, s1_ref[...], preferred_element_type=f32)
    p1 = p1.astype(bf16)

    p1r, p1l = _shift_lr(p1, 32)
    ps = jnp.concatenate([p1r, p1, p1l], axis=0)

    y2 = jnp.dot(m2_ref[...], ps, preferred_element_type=f32)
    y2 = jnp.maximum(y2 + b2_ref[...], 0.0)

    tr2 = _pool_rows(y2, scratches)
    u2 = _col_neighbor_max(tr2).astype(bf16)
    return jnp.dot(u2, s2_ref[...], preferred_element_type=f32).astype(bf16)


def _conv_stack_kernel(x_ref, m1_ref, b1_ref, m2_ref, b2_ref, s1_ref, s2_ref,
                       out_ref, *scratches):
    ns = len(scratches) // _CHAINS
    for u in range(_CHAINS):
        out_ref[0, u, :, :] = _conv_chain(
            x_ref[0, u, :, :], m1_ref, b1_ref, m2_ref, b2_ref, s1_ref, s2_ref,
            scratches[u * ns:(u + 1) * ns])


def _fc_stack_kernel(x_ref, w1_ref, b1_ref, w2_ref, b2_ref, w3_ref, b3_ref,
                     o_ref):
    f32 = jnp.float32
    h = jnp.dot(x_ref[...], w1_ref[...], preferred_element_type=f32)
    h = jnp.maximum(h + b1_ref[...], 0.0)
    h = jnp.dot(h, w2_ref[...], preferred_element_type=f32)
    h = jnp.maximum(h + b2_ref[...], 0.0)
    o = jnp.dot(h, w3_ref[...], preferred_element_type=f32) + b3_ref[...]
    o_ref[...] = o.astype(o_ref.dtype)


def kernel(x, m1_0, m1_1, m1_2, c1_0, c1_2, b1s, re1, ro1, pe1, po1,
           m2_0, m2_1, m2_2, c2_0, c2_2, b2s, re2, ro2, pe2, po2,
           fc1_w, fc1_b, fc2_w, fc2_b, fc3_w, fc3_b):
    f32 = jnp.float32
    bf16 = jnp.bfloat16

    N = x.shape[0]
    assert x.shape[1:] == (3, 64, 64), x.shape
    G = _PACK * _CHAINS
    Np = _round_up(N, G)
    x = x.astype(bf16)
    if Np != N:
        x = jnp.pad(x, ((0, Np - N), (0, 0), (0, 0), (0, 0)))
    Nb = Np // G

    xp = x.reshape(Nb, _CHAINS, _PACK, 3, 64, 64).transpose(0, 1, 3, 4, 2, 5)
    xp = xp.reshape(Nb, _CHAINS, 3 * 64, _PACK * 64)

    m1 = jnp.concatenate([m1_0, m1_1, m1_2], axis=1).astype(bf16)
    m2 = jnp.concatenate([m2_0, m2_1, m2_2], axis=1).astype(bf16)
    s1 = jnp.asarray(_col_compact_sel(64, _PACK), bf16)
    s2 = jnp.asarray(_col_compact_sel(32, _PACK), bf16)

    nsc = _PACK * 64 // 128
    conv_out = pl.pallas_call(
        _conv_stack_kernel,
        out_shape=jax.ShapeDtypeStruct((Nb, _CHAINS, 12 * 16, _PACK * 16),
                                       bf16),
        grid=(Nb,),
        in_specs=[
            pl.BlockSpec((1, _CHAINS, 3 * 64, _PACK * 64),
                         lambda i: (i, 0, 0, 0)),
            pl.BlockSpec(m1.shape, lambda i: (0, 0)),
            pl.BlockSpec(b1s.shape, lambda i: (0, 0)),
            pl.BlockSpec(m2.shape, lambda i: (0, 0)),
            pl.BlockSpec(b2s.shape, lambda i: (0, 0)),
            pl.BlockSpec(s1.shape, lambda i: (0, 0)),
            pl.BlockSpec(s2.shape, lambda i: (0, 0)),
        ],
        out_specs=pl.BlockSpec((1, _CHAINS, 12 * 16, _PACK * 16),
                               lambda i: (i, 0, 0, 0)),
        scratch_shapes=[pltpu.VMEM((6 * 64, 128), f32)
                        for _ in range(_CHAINS * nsc)],
        compiler_params=pltpu.CompilerParams(dimension_semantics=("parallel",)),
    )(xp, m1, b1s.astype(f32), m2, b2s.astype(f32), s1, s2)

    feat = conv_out.reshape(Nb, _CHAINS, 12, 16, _PACK, 16)
    feat = feat.transpose(0, 1, 4, 2, 3, 5)
    flat = feat.reshape(Np, 12 * 16 * 16)[:N]

    K = flat.shape[1]
    n1 = fc1_w.shape[1]
    n2 = fc2_w.shape[1]
    n3 = fc3_w.shape[1]

    TB = min(128, _round_up(N, 8))
    Nf = _round_up(N, TB)
    if Nf != N:
        flat = jnp.pad(flat, ((0, Nf - N), (0, 0)))

    out = pl.pallas_call(
        _fc_stack_kernel,
        out_shape=jax.ShapeDtypeStruct((Nf, n3), f32),
        grid=(Nf // TB,),
        in_specs=[
            pl.BlockSpec((TB, K), lambda i: (i, 0)),
            pl.BlockSpec((K, n1), lambda i: (0, 0)),
            pl.BlockSpec((1, n1), lambda i: (0, 0)),
            pl.BlockSpec((n1, n2), lambda i: (0, 0)),
            pl.BlockSpec((1, n2), lambda i: (0, 0)),
            pl.BlockSpec((n2, n3), lambda i: (0, 0)),
            pl.BlockSpec((1, n3), lambda i: (0, 0)),
        ],
        out_specs=pl.BlockSpec((TB, n3), lambda i: (i, 0)),
        compiler_params=pltpu.CompilerParams(dimension_semantics=("parallel",)),
    )(flat, fc1_w.astype(bf16), fc1_b.astype(f32),
      fc2_w.astype(f32), fc2_b.astype(f32),
      fc3_w.astype(f32), fc3_b.astype(f32))
    return out[:N]


# speedup vs baseline: 1.4481x; 1.0539x over previous
import numpy as np
import jax
import jax.numpy as jnp
from jax.experimental import pallas as pl
from jax.experimental.pallas import tpu as pltpu

_PACK = 8


def _round_up(n, m):
    return ((n + m - 1) // m) * m


def _col_compact_sel(w, pack):
    S = np.zeros((w, w // 2), np.float32)
    S[2 * np.arange(w // 2), np.arange(w // 2)] = 1.0
    return np.kron(np.eye(pack, dtype=np.float32), S)


def _shift_lr(x, img_w):
    R, L = x.shape
    z = jnp.zeros((R, 1), x.dtype)
    xl = jnp.concatenate([x[:, 1:], z], axis=1)
    xr = jnp.concatenate([z, x[:, :-1]], axis=1)
    col = jax.lax.broadcasted_iota(jnp.int32, (1, L), 1) % img_w
    xl = jnp.where(col == img_w - 1, jnp.zeros((), x.dtype), xl)
    xr = jnp.where(col == 0, jnp.zeros((), x.dtype), xr)
    return xr, xl


def _pool2x2(y, scratches, sel_ref):
    R, L = y.shape
    t = jnp.maximum(y, jnp.concatenate([y[1:, :], y[:1, :]], axis=0))
    nc = L // 128
    for c in range(nc):
        scratches[c][:R, :] = t[:, c * 128:(c + 1) * 128]
    tr = jnp.concatenate(
        [scratches[c][pl.ds(0, R // 2, 2), :] for c in range(nc)], axis=1)
    u = jnp.maximum(tr, jnp.concatenate([tr[:, 1:], tr[:, :1]], axis=1))
    return jnp.dot(u.astype(sel_ref.dtype), sel_ref[:L, :],
                   preferred_element_type=jnp.float32)


def _conv_stack_kernel(x_ref, m1_ref, b1_ref, m2_ref, b2_ref,
                       s1_ref, s2_ref, out_ref, *scratches):
    f32 = jnp.float32
    bf16 = jnp.bfloat16

    x = x_ref[0, :, :]
    xr, xl = _shift_lr(x, 64)
    xs = jnp.concatenate([xr, x, xl], axis=0).astype(bf16)

    y = jnp.dot(m1_ref[...], xs, preferred_element_type=f32)
    p1 = _pool2x2(y, scratches, s1_ref)
    p1 = jnp.maximum(p1 + b1_ref[...], 0.0)

    p1r, p1l = _shift_lr(p1, 32)
    ps = jnp.concatenate([p1r, p1, p1l], axis=0).astype(bf16)

    y2 = jnp.dot(m2_ref[...], ps, preferred_element_type=f32)
    p2 = _pool2x2(y2, scratches, s2_ref)
    p2 = jnp.maximum(p2 + b2_ref[...], 0.0)

    out_ref[0, :, :] = p2.astype(out_ref.dtype)


def _fc_stack_kernel(x_ref, w1_ref, b1_ref, w2_ref, b2_ref, w3_ref, b3_ref,
                     o_ref):
    f32 = jnp.float32
    h = jnp.dot(x_ref[...], w1_ref[...], preferred_element_type=f32)
    h = jnp.maximum(h + b1_ref[...], 0.0)
    h = jnp.dot(h, w2_ref[...], preferred_element_type=f32)
    h = jnp.maximum(h + b2_ref[...], 0.0)
    o = jnp.dot(h, w3_ref[...], preferred_element_type=f32) + b3_ref[...]
    o_ref[...] = o.astype(o_ref.dtype)


def kernel(x, m1_0, m1_1, m1_2, c1_0, c1_2, b1s, re1, ro1, pe1, po1,
           m2_0, m2_1, m2_2, c2_0, c2_2, b2s, re2, ro2, pe2, po2,
           fc1_w, fc1_b, fc2_w, fc2_b, fc3_w, fc3_b):
    f32 = jnp.float32
    bf16 = jnp.bfloat16

    N = x.shape[0]
    assert x.shape[1:] == (3, 64, 64), x.shape
    Np = _round_up(N, _PACK)
    x = x.astype(f32)
    if Np != N:
        x = jnp.pad(x, ((0, Np - N), (0, 0), (0, 0), (0, 0)))
    Nb = Np // _PACK

    xp = x.reshape(Nb, _PACK, 3, 64, 64).transpose(0, 2, 3, 1, 4)
    xp = xp.reshape(Nb, 3 * 64, _PACK * 64)

    m1 = jnp.concatenate([m1_0, m1_1, m1_2], axis=1).astype(bf16)
    m2 = jnp.concatenate([m2_0, m2_1, m2_2], axis=1).astype(bf16)
    s1 = jnp.asarray(_col_compact_sel(64, _PACK), bf16)
    s2 = jnp.asarray(_col_compact_sel(32, _PACK), bf16)
    b1p = b1s.astype(f32)[::2]
    b2p = b2s.astype(f32)[::2]

    conv_out = pl.pallas_call(
        _conv_stack_kernel,
        out_shape=jax.ShapeDtypeStruct((Nb, 12 * 16, _PACK * 16), bf16),
        grid=(Nb,),
        in_specs=[
            pl.BlockSpec((1, 3 * 64, _PACK * 64), lambda i: (i, 0, 0)),
            pl.BlockSpec(m1.shape, lambda i: (0, 0)),
            pl.BlockSpec(b1p.shape, lambda i: (0, 0)),
            pl.BlockSpec(m2.shape, lambda i: (0, 0)),
            pl.BlockSpec(b2p.shape, lambda i: (0, 0)),
            pl.BlockSpec(s1.shape, lambda i: (0, 0)),
            pl.BlockSpec(s2.shape, lambda i: (0, 0)),
        ],
        out_specs=pl.BlockSpec((1, 12 * 16, _PACK * 16), lambda i: (i, 0, 0)),
        scratch_shapes=[pltpu.VMEM((6 * 64, 128), f32)
                        for _ in range(_PACK * 64 // 128)],
        compiler_params=pltpu.CompilerParams(dimension_semantics=("parallel",)),
    )(xp, m1, b1p, m2, b2p, s1, s2)

    feat = conv_out.reshape(Nb, 12, 16, _PACK, 16).transpose(0, 3, 1, 2, 4)
    flat = feat.reshape(Np, 12 * 16 * 16)[:N]

    K = flat.shape[1]
    n1 = fc1_w.shape[1]
    n2 = fc2_w.shape[1]
    n3 = fc3_w.shape[1]

    TB = min(128, _round_up(N, 8))
    Nf = _round_up(N, TB)
    if Nf != N:
        flat = jnp.pad(flat, ((0, Nf - N), (0, 0)))

    out = pl.pallas_call(
        _fc_stack_kernel,
        out_shape=jax.ShapeDtypeStruct((Nf, n3), f32),
        grid=(Nf // TB,),
        in_specs=[
            pl.BlockSpec((TB, K), lambda i: (i, 0)),
            pl.BlockSpec((K, n1), lambda i: (0, 0)),
            pl.BlockSpec((1, n1), lambda i: (0, 0)),
            pl.BlockSpec((n1, n2), lambda i: (0, 0)),
            pl.BlockSpec((1, n2), lambda i: (0, 0)),
            pl.BlockSpec((n2, n3), lambda i: (0, 0)),
            pl.BlockSpec((1, n3), lambda i: (0, 0)),
        ],
        out_specs=pl.BlockSpec((TB, n3), lambda i: (i, 0)),
        compiler_params=pltpu.CompilerParams(dimension_semantics=("parallel",)),
    )(flat, fc1_w.astype(bf16), fc1_b.astype(f32),
      fc2_w.astype(f32), fc2_b.astype(f32),
      fc3_w.astype(f32), fc3_b.astype(f32))
    return out[:N]


# speedup vs baseline: 1.4495x; 1.0010x over previous
import numpy as np
import jax
import jax.numpy as jnp
from jax.experimental import pallas as pl
from jax.experimental.pallas import tpu as pltpu

_PACK = 8


def _round_up(n, m):
    return ((n + m - 1) // m) * m


def _col_compact_sel(w, pack):
    S = np.zeros((w, w // 2), np.float32)
    S[2 * np.arange(w // 2), np.arange(w // 2)] = 1.0
    return np.kron(np.eye(pack, dtype=np.float32), S)


def _shift_lr(x, img_w):
    R, L = x.shape
    z = jnp.zeros((R, 1), x.dtype)
    xl = jnp.concatenate([x[:, 1:], z], axis=1)
    xr = jnp.concatenate([z, x[:, :-1]], axis=1)
    col = jax.lax.broadcasted_iota(jnp.int32, (1, L), 1) % img_w
    xl = jnp.where(col == img_w - 1, jnp.zeros((), x.dtype), xl)
    xr = jnp.where(col == 0, jnp.zeros((), x.dtype), xr)
    return xr, xl


def _pool2x2(y, b_ref, scratches, sel_ref):
    R, L = y.shape
    t = jnp.maximum(y, jnp.concatenate([y[1:, :], y[:1, :]], axis=0))
    nc = L // 128
    for c in range(nc):
        scratches[c][:R, :] = t[:, c * 128:(c + 1) * 128]
    tr = jnp.concatenate(
        [scratches[c][pl.ds(0, R // 2, 2), :] for c in range(nc)], axis=1)
    tr = tr + b_ref[...]
    u = jnp.maximum(tr, jnp.concatenate([tr[:, 1:], tr[:, :1]], axis=1))
    p = jnp.dot(u.astype(sel_ref.dtype), sel_ref[:L, :],
                preferred_element_type=jnp.float32)
    return jnp.maximum(p, 0.0)


def _conv_stack_kernel(x_ref, m1_ref, b1_ref, m2_ref, b2_ref,
                       s1_ref, s2_ref, out_ref, *scratches):
    f32 = jnp.float32
    bf16 = jnp.bfloat16

    x = x_ref[0, :, :]
    xr, xl = _shift_lr(x, 64)
    xs = jnp.concatenate([xr, x, xl], axis=0).astype(bf16)

    y = jnp.dot(m1_ref[...], xs, preferred_element_type=f32)
    p1 = _pool2x2(y, b1_ref, scratches, s1_ref)

    p1r, p1l = _shift_lr(p1, 32)
    ps = jnp.concatenate([p1r, p1, p1l], axis=0).astype(bf16)

    y2 = jnp.dot(m2_ref[...], ps, preferred_element_type=f32)
    p2 = _pool2x2(y2, b2_ref, scratches, s2_ref)

    out_ref[0, :, :] = p2.astype(out_ref.dtype)


def _fc_stack_kernel(x_ref, w1_ref, b1_ref, w2_ref, b2_ref, w3_ref, b3_ref,
                     o_ref):
    f32 = jnp.float32
    h = jnp.dot(x_ref[...], w1_ref[...], preferred_element_type=f32)
    h = jnp.maximum(h + b1_ref[...], 0.0)
    h = jnp.dot(h, w2_ref[...], preferred_element_type=f32)
    h = jnp.maximum(h + b2_ref[...], 0.0)
    o = jnp.dot(h, w3_ref[...], preferred_element_type=f32) + b3_ref[...]
    o_ref[...] = o.astype(o_ref.dtype)


def kernel(x, m1_0, m1_1, m1_2, c1_0, c1_2, b1s, re1, ro1, pe1, po1,
           m2_0, m2_1, m2_2, c2_0, c2_2, b2s, re2, ro2, pe2, po2,
           fc1_w, fc1_b, fc2_w, fc2_b, fc3_w, fc3_b):
    f32 = jnp.float32
    bf16 = jnp.bfloat16

    N = x.shape[0]
    assert x.shape[1:] == (3, 64, 64), x.shape
    Np = _round_up(N, _PACK)
    x = x.astype(f32)
    if Np != N:
        x = jnp.pad(x, ((0, Np - N), (0, 0), (0, 0), (0, 0)))
    Nb = Np // _PACK

    xp = x.reshape(Nb, _PACK, 3, 64, 64).transpose(0, 2, 3, 1, 4)
    xp = xp.reshape(Nb, 3 * 64, _PACK * 64)

    m1 = jnp.concatenate([m1_0, m1_1, m1_2], axis=1).astype(bf16)
    m2 = jnp.concatenate([m2_0, m2_1, m2_2], axis=1).astype(bf16)
    s1 = jnp.asarray(_col_compact_sel(64, _PACK), bf16)
    s2 = jnp.asarray(_col_compact_sel(32, _PACK), bf16)
    b1p = b1s.astype(f32)[::2]
    b2p = b2s.astype(f32)[::2]

    conv_out = pl.pallas_call(
        _conv_stack_kernel,
        out_shape=jax.ShapeDtypeStruct((Nb, 12 * 16, _PACK * 16), bf16),
        grid=(Nb,),
        in_specs=[
            pl.BlockSpec((1, 3 * 64, _PACK * 64), lambda i: (i, 0, 0)),
            pl.BlockSpec(m1.shape, lambda i: (0, 0)),
            pl.BlockSpec(b1p.shape, lambda i: (0, 0)),
            pl.BlockSpec(m2.shape, lambda i: (0, 0)),
            pl.BlockSpec(b2p.shape, lambda i: (0, 0)),
            pl.BlockSpec(s1.shape, lambda i: (0, 0)),
            pl.BlockSpec(s2.shape, lambda i: (0, 0)),
        ],
        out_specs=pl.BlockSpec((1, 12 * 16, _PACK * 16), lambda i: (i, 0, 0)),
        scratch_shapes=[pltpu.VMEM((6 * 64, 128), f32)
                        for _ in range(_PACK * 64 // 128)],
        compiler_params=pltpu.CompilerParams(dimension_semantics=("parallel",)),
    )(xp, m1, b1p, m2, b2p, s1, s2)

    feat = conv_out.reshape(Nb, 12, 16, _PACK, 16).transpose(0, 3, 1, 2, 4)
    flat = feat.reshape(Np, 12 * 16 * 16)[:N]

    K = flat.shape[1]
    n1 = fc1_w.shape[1]
    n2 = fc2_w.shape[1]
    n3 = fc3_w.shape[1]

    TB = min(128, _round_up(N, 8))
    Nf = _round_up(N, TB)
    if Nf != N:
        flat = jnp.pad(flat, ((0, Nf - N), (0, 0)))

    out = pl.pallas_call(
        _fc_stack_kernel,
        out_shape=jax.ShapeDtypeStruct((Nf, n3), f32),
        grid=(Nf // TB,),
        in_specs=[
            pl.BlockSpec((TB, K), lambda i: (i, 0)),
            pl.BlockSpec((K, n1), lambda i: (0, 0)),
            pl.BlockSpec((1, n1), lambda i: (0, 0)),
            pl.BlockSpec((n1, n2), lambda i: (0, 0)),
            pl.BlockSpec((1, n2), lambda i: (0, 0)),
            pl.BlockSpec((n2, n3), lambda i: (0, 0)),
            pl.BlockSpec((1, n3), lambda i: (0, 0)),
        ],
        out_specs=pl.BlockSpec((TB, n3), lambda i: (i, 0)),
        compiler_params=pltpu.CompilerParams(dimension_semantics=("parallel",)),
    )(flat, fc1_w.astype(bf16), fc1_b.astype(f32),
      fc2_w.astype(f32), fc2_b.astype(f32),
      fc3_w.astype(f32), fc3_b.astype(f32))
    return out[:N]


# speedup vs baseline: 1.8555x; 1.2801x over previous
import numpy as np
import jax
import jax.numpy as jnp
from jax.experimental import pallas as pl
from jax.experimental.pallas import tpu as pltpu

_PACK = 8


def _round_up(n, m):
    return ((n + m - 1) // m) * m


def _col_compact_sel(w, pack):
    S = np.zeros((w, w // 2), np.float32)
    S[2 * np.arange(w // 2), np.arange(w // 2)] = 1.0
    return np.kron(np.eye(pack, dtype=np.float32), S)


def _fc1_perm():
    j = np.arange(16)[:, None, None]
    c = np.arange(12)[None, :, None]
    i = np.arange(16)[None, None, :]
    return (c * 256 + i * 16 + j).reshape(-1)


def _shift_lr(x, img_w):
    R, L = x.shape
    z = jnp.zeros((R, 1), x.dtype)
    xl = jnp.concatenate([x[:, 1:], z], axis=1)
    xr = jnp.concatenate([z, x[:, :-1]], axis=1)
    col = jax.lax.broadcasted_iota(jnp.int32, (1, L), 1) % img_w
    xl = jnp.where(col == img_w - 1, jnp.zeros((), x.dtype), xl)
    xr = jnp.where(col == 0, jnp.zeros((), x.dtype), xr)
    return xr, xl


def _pool2x2(y, b_ref, scratches, sel_ref):
    R, L = y.shape
    t = jnp.maximum(y, jnp.concatenate([y[1:, :], y[:1, :]], axis=0))
    nc = L // 128
    for c in range(nc):
        scratches[c][:R, :] = t[:, c * 128:(c + 1) * 128]
    tr = jnp.concatenate(
        [scratches[c][pl.ds(0, R // 2, 2), :] for c in range(nc)], axis=1)
    tr = tr + b_ref[...]
    u = jnp.maximum(tr, jnp.concatenate([tr[:, 1:], tr[:, :1]], axis=1))
    p = jnp.dot(u.astype(sel_ref.dtype), sel_ref[:L, :],
                preferred_element_type=jnp.float32)
    return jnp.maximum(p, 0.0)


def _conv_stack_kernel(x_ref, m1_ref, b1_ref, m2_ref, b2_ref,
                       s1_ref, s2_ref, out_ref, xscr, *scratches):
    f32 = jnp.float32
    bf16 = jnp.bfloat16

    for q in range(_PACK // 2):
        a = x_ref[2 * q].reshape(3 * 64, 64)
        b = x_ref[2 * q + 1].reshape(3 * 64, 64)
        xscr[:, 128 * q:128 * (q + 1)] = jnp.concatenate([a, b], axis=1)
    x = xscr[:, :]

    xr, xl = _shift_lr(x, 64)
    xs = jnp.concatenate([xr, x, xl], axis=0).astype(bf16)

    y = jnp.dot(m1_ref[...], xs, preferred_element_type=f32)
    p1 = _pool2x2(y, b1_ref, scratches, s1_ref)

    p1r, p1l = _shift_lr(p1, 32)
    ps = jnp.concatenate([p1r, p1, p1l], axis=0).astype(bf16)

    y2 = jnp.dot(m2_ref[...], ps, preferred_element_type=f32)
    p2 = _pool2x2(y2, b2_ref, scratches, s2_ref)

    out_ref[0, :, :] = jnp.transpose(p2).astype(out_ref.dtype)


def _fc_stack_kernel(x_ref, w1_ref, b1_ref, w2_ref, b2_ref, w3_ref, b3_ref,
                     o_ref):
    f32 = jnp.float32
    h = jnp.dot(x_ref[...], w1_ref[...], preferred_element_type=f32)
    h = jnp.maximum(h + b1_ref[...], 0.0)
    h = jnp.dot(h, w2_ref[...], preferred_element_type=f32)
    h = jnp.maximum(h + b2_ref[...], 0.0)
    o = jnp.dot(h, w3_ref[...], preferred_element_type=f32) + b3_ref[...]
    o_ref[...] = o.astype(o_ref.dtype)


def kernel(x, m1_0, m1_1, m1_2, c1_0, c1_2, b1s, re1, ro1, pe1, po1,
           m2_0, m2_1, m2_2, c2_0, c2_2, b2s, re2, ro2, pe2, po2,
           fc1_w, fc1_b, fc2_w, fc2_b, fc3_w, fc3_b):
    f32 = jnp.float32
    bf16 = jnp.bfloat16

    N = x.shape[0]
    assert x.shape[1:] == (3, 64, 64), x.shape
    Np = _round_up(N, _PACK)
    x = x.astype(f32)
    if Np != N:
        x = jnp.pad(x, ((0, Np - N), (0, 0), (0, 0), (0, 0)))
    Nb = Np // _PACK

    m1 = jnp.concatenate([m1_0, m1_1, m1_2], axis=1).astype(bf16)
    m2 = jnp.concatenate([m2_0, m2_1, m2_2], axis=1).astype(bf16)
    s1 = jnp.asarray(_col_compact_sel(64, _PACK), bf16)
    s2 = jnp.asarray(_col_compact_sel(32, _PACK), bf16)
    b1p = b1s.astype(f32)[::2]
    b2p = b2s.astype(f32)[::2]

    conv_out = pl.pallas_call(
        _conv_stack_kernel,
        out_shape=jax.ShapeDtypeStruct((Nb, _PACK * 16, 12 * 16), bf16),
        grid=(Nb,),
        in_specs=[
            pl.BlockSpec((_PACK, 3, 64, 64), lambda i: (i, 0, 0, 0)),
            pl.BlockSpec(m1.shape, lambda i: (0, 0)),
            pl.BlockSpec(b1p.shape, lambda i: (0, 0)),
            pl.BlockSpec(m2.shape, lambda i: (0, 0)),
            pl.BlockSpec(b2p.shape, lambda i: (0, 0)),
            pl.BlockSpec(s1.shape, lambda i: (0, 0)),
            pl.BlockSpec(s2.shape, lambda i: (0, 0)),
        ],
        out_specs=pl.BlockSpec((1, _PACK * 16, 12 * 16), lambda i: (i, 0, 0)),
        scratch_shapes=[pltpu.VMEM((3 * 64, _PACK * 64), f32)]
        + [pltpu.VMEM((6 * 64, 128), f32) for _ in range(_PACK * 64 // 128)],
        compiler_params=pltpu.CompilerParams(dimension_semantics=("parallel",)),
    )(x, m1, b1p, m2, b2p, s1, s2)

    flat = conv_out.reshape(Np, 16 * 192)[:N]
    w1p = fc1_w[jnp.asarray(_fc1_perm()), :].astype(bf16)

    K = flat.shape[1]
    n1 = fc1_w.shape[1]
    n2 = fc2_w.shape[1]
    n3 = fc3_w.shape[1]

    TB = min(128, _round_up(N, 8))
    Nf = _round_up(N, TB)
    if Nf != N:
        flat = jnp.pad(flat, ((0, Nf - N), (0, 0)))

    out = pl.pallas_call(
        _fc_stack_kernel,
        out_shape=jax.ShapeDtypeStruct((Nf, n3), f32),
        grid=(Nf // TB,),
        in_specs=[
            pl.BlockSpec((TB, K), lambda i: (i, 0)),
            pl.BlockSpec((K, n1), lambda i: (0, 0)),
            pl.BlockSpec((1, n1), lambda i: (0, 0)),
            pl.BlockSpec((n1, n2), lambda i: (0, 0)),
            pl.BlockSpec((1, n2), lambda i: (0, 0)),
            pl.BlockSpec((n2, n3), lambda i: (0, 0)),
            pl.BlockSpec((1, n3), lambda i: (0, 0)),
        ],
        out_specs=pl.BlockSpec((TB, n3), lambda i: (i, 0)),
        compiler_params=pltpu.CompilerParams(dimension_semantics=("parallel",)),
    )(flat, w1p, fc1_b.astype(f32),
      fc2_w.astype(f32), fc2_b.astype(f32),
      fc3_w.astype(f32), fc3_b.astype(f32))
    return out[:N]


# speedup vs baseline: 2.0185x; 1.0878x over previous
import numpy as np
import jax
import jax.numpy as jnp
from jax.experimental import pallas as pl
from jax.experimental.pallas import tpu as pltpu

_PACK = 8
_CHAINS = 2


def _round_up(n, m):
    return ((n + m - 1) // m) * m


def _col_compact_sel(w, pack):
    S = np.zeros((w, w // 2), np.float32)
    S[2 * np.arange(w // 2), np.arange(w // 2)] = 1.0
    return np.kron(np.eye(pack, dtype=np.float32), S)


def _fc1_perm():
    j = np.arange(16)[:, None, None]
    c = np.arange(12)[None, :, None]
    i = np.arange(16)[None, None, :]
    return (c * 256 + i * 16 + j).reshape(-1)


def _shift_lr(x, img_w):
    R, L = x.shape
    z = jnp.zeros((R, 1), x.dtype)
    xl = jnp.concatenate([x[:, 1:], z], axis=1)
    xr = jnp.concatenate([z, x[:, :-1]], axis=1)
    col = jax.lax.broadcasted_iota(jnp.int32, (1, L), 1) % img_w
    xl = jnp.where(col == img_w - 1, jnp.zeros((), x.dtype), xl)
    xr = jnp.where(col == 0, jnp.zeros((), x.dtype), xr)
    return xr, xl


def _pool2x2(y, b_ref, sel_ref):
    R, L = y.shape
    tr = jnp.maximum(y[:R // 2, :], y[R // 2:, :]) + b_ref[...]
    u = jnp.maximum(tr, jnp.concatenate([tr[:, 1:], tr[:, :1]], axis=1))
    p = jnp.dot(u.astype(sel_ref.dtype), sel_ref[:L, :],
                preferred_element_type=jnp.float32)
    return jnp.maximum(p, 0.0)


def _row_par_perm(C, Ho):
    par = np.arange(2)[:, None, None]
    c = np.arange(C)[None, :, None]
    i2 = np.arange(Ho)[None, None, :]
    return (c * 2 * Ho + 2 * i2 + par).reshape(-1)


def _conv_chain(x_ref, base, m1_ref, b1_ref, m2_ref, b2_ref,
                s1_ref, s2_ref, out_ref, xscr):
    f32 = jnp.float32
    bf16 = jnp.bfloat16

    for q in range(_PACK // 2):
        a = x_ref[base + 2 * q].reshape(3 * 64, 64)
        b = x_ref[base + 2 * q + 1].reshape(3 * 64, 64)
        xscr[:, 128 * q:128 * (q + 1)] = jnp.concatenate([a, b], axis=1)
    x = xscr[:, :]

    xr, xl = _shift_lr(x, 64)
    xs = jnp.concatenate([xr, x, xl], axis=0).astype(bf16)

    y = jnp.dot(m1_ref[...], xs, preferred_element_type=f32)
    p1 = _pool2x2(y, b1_ref, s1_ref)

    p1r, p1l = _shift_lr(p1, 32)
    ps = jnp.concatenate([p1r, p1, p1l], axis=0).astype(bf16)

    y2 = jnp.dot(m2_ref[...], ps, preferred_element_type=f32)
    p2 = _pool2x2(y2, b2_ref, s2_ref)

    out_ref[0, base * 16:(base + _PACK) * 16, :] = (
        jnp.transpose(p2).astype(out_ref.dtype))


def _conv_stack_kernel(x_ref, m1_ref, b1_ref, m2_ref, b2_ref,
                       s1_ref, s2_ref, out_ref, *xscrs):
    for u in range(_CHAINS):
        _conv_chain(x_ref, u * _PACK, m1_ref, b1_ref, m2_ref, b2_ref,
                    s1_ref, s2_ref, out_ref, xscrs[u])


def _fc_stack_kernel(x_ref, w1_ref, b1_ref, w2_ref, b2_ref, w3_ref, b3_ref,
                     o_ref):
    f32 = jnp.float32
    h = jnp.dot(x_ref[...], w1_ref[...], preferred_element_type=f32)
    h = jnp.maximum(h + b1_ref[...], 0.0)
    h = jnp.dot(h, w2_ref[...], preferred_element_type=f32)
    h = jnp.maximum(h + b2_ref[...], 0.0)
    o = jnp.dot(h, w3_ref[...], preferred_element_type=f32) + b3_ref[...]
    o_ref[...] = o.astype(o_ref.dtype)


def kernel(x, m1_0, m1_1, m1_2, c1_0, c1_2, b1s, re1, ro1, pe1, po1,
           m2_0, m2_1, m2_2, c2_0, c2_2, b2s, re2, ro2, pe2, po2,
           fc1_w, fc1_b, fc2_w, fc2_b, fc3_w, fc3_b):
    f32 = jnp.float32
    bf16 = jnp.bfloat16

    N = x.shape[0]
    assert x.shape[1:] == (3, 64, 64), x.shape
    G = _PACK * _CHAINS
    Np = _round_up(N, G)
    x = x.astype(f32)
    if Np != N:
        x = jnp.pad(x, ((0, Np - N), (0, 0), (0, 0), (0, 0)))
    Nb = Np // G

    m1 = jnp.concatenate([m1_0, m1_1, m1_2], axis=1)
    m1 = m1[jnp.asarray(_row_par_perm(6, 32)), :].astype(bf16)
    m2 = jnp.concatenate([m2_0, m2_1, m2_2], axis=1)
    m2 = m2[jnp.asarray(_row_par_perm(12, 16)), :].astype(bf16)
    s1 = jnp.asarray(_col_compact_sel(64, _PACK), bf16)
    s2 = jnp.asarray(_col_compact_sel(32, _PACK), bf16)
    b1p = b1s.astype(f32)[::2]
    b2p = b2s.astype(f32)[::2]

    conv_out = pl.pallas_call(
        _conv_stack_kernel,
        out_shape=jax.ShapeDtypeStruct((Nb, G * 16, 12 * 16), bf16),
        grid=(Nb,),
        in_specs=[
            pl.BlockSpec((G, 3, 64, 64), lambda i: (i, 0, 0, 0)),
            pl.BlockSpec(m1.shape, lambda i: (0, 0)),
            pl.BlockSpec(b1p.shape, lambda i: (0, 0)),
            pl.BlockSpec(m2.shape, lambda i: (0, 0)),
            pl.BlockSpec(b2p.shape, lambda i: (0, 0)),
            pl.BlockSpec(s1.shape, lambda i: (0, 0)),
            pl.BlockSpec(s2.shape, lambda i: (0, 0)),
        ],
        out_specs=pl.BlockSpec((1, G * 16, 12 * 16), lambda i: (i, 0, 0)),
        scratch_shapes=[pltpu.VMEM((3 * 64, _PACK * 64), f32)
                        for _ in range(_CHAINS)],
        compiler_params=pltpu.CompilerParams(dimension_semantics=("parallel",)),
    )(x, m1, b1p, m2, b2p, s1, s2)

    flat = conv_out.reshape(Np, 16 * 192)[:N]
    w1p = fc1_w[jnp.asarray(_fc1_perm()), :].astype(bf16)

    K = flat.shape[1]
    n1 = fc1_w.shape[1]
    n2 = fc2_w.shape[1]
    n3 = fc3_w.shape[1]

    TB = min(128, _round_up(N, 8))
    Nf = _round_up(N, TB)
    if Nf != N:
        flat = jnp.pad(flat, ((0, Nf - N), (0, 0)))

    out = pl.pallas_call(
        _fc_stack_kernel,
        out_shape=jax.ShapeDtypeStruct((Nf, n3), f32),
        grid=(Nf // TB,),
        in_specs=[
            pl.BlockSpec((TB, K), lambda i: (i, 0)),
            pl.BlockSpec((K, n1), lambda i: (0, 0)),
            pl.BlockSpec((1, n1), lambda i: (0, 0)),
            pl.BlockSpec((n1, n2), lambda i: (0, 0)),
            pl.BlockSpec((1, n2), lambda i: (0, 0)),
            pl.BlockSpec((n2, n3), lambda i: (0, 0)),
            pl.BlockSpec((1, n3), lambda i: (0, 0)),
        ],
        out_specs=pl.BlockSpec((TB, n3), lambda i: (i, 0)),
        compiler_params=pltpu.CompilerParams(dimension_semantics=("parallel",)),
    )(flat, w1p, fc1_b.astype(f32),
      fc2_w.astype(f32), fc2_b.astype(f32),
      fc3_w.astype(f32), fc3_b.astype(f32))
    return out[:N]


# speedup vs baseline: 2.2773x; 1.1282x over previous
import numpy as np
import jax
import jax.numpy as jnp
from jax.experimental import pallas as pl
from jax.experimental.pallas import tpu as pltpu

_PACK = 8
_CHAINS = 2


def _round_up(n, m):
    return ((n + m - 1) // m) * m


def _col_compact_sel(w, pack):
    S = np.zeros((w, w // 2), np.float32)
    S[2 * np.arange(w // 2), np.arange(w // 2)] = 1.0
    return np.kron(np.eye(pack, dtype=np.float32), S)


def _fc1_perm():
    j = np.arange(16)[:, None, None]
    c = np.arange(12)[None, :, None]
    i = np.arange(16)[None, None, :]
    return (c * 256 + i * 16 + j).reshape(-1)


def _shift_lr(x, img_w):
    R, L = x.shape
    z = jnp.zeros((R, 1), x.dtype)
    xl = jnp.concatenate([x[:, 1:], z], axis=1)
    xr = jnp.concatenate([z, x[:, :-1]], axis=1)
    col = jax.lax.broadcasted_iota(jnp.int32, (1, L), 1) % img_w
    xl = jnp.where(col == img_w - 1, jnp.zeros((), x.dtype), xl)
    xr = jnp.where(col == 0, jnp.zeros((), x.dtype), xr)
    return xr, xl


def _pool2x2(y, b_ref, sel_ref):
    R, L = y.shape
    tr = jnp.maximum(y[:R // 2, :], y[R // 2:, :]) + b_ref[...]
    u = jnp.maximum(tr, jnp.concatenate([tr[:, 1:], tr[:, :1]], axis=1))
    p = jnp.dot(u.astype(sel_ref.dtype), sel_ref[:L, :],
                preferred_element_type=jnp.float32)
    return jnp.maximum(p, 0.0)


def _row_par_perm(C, Ho):
    par = np.arange(2)[:, None, None]
    c = np.arange(C)[None, :, None]
    i2 = np.arange(Ho)[None, None, :]
    return (c * 2 * Ho + 2 * i2 + par).reshape(-1)


def _conv_chain(x_ref, base, m1_ref, b1_ref, m2_ref, b2_ref,
                s1_ref, s2_ref, out_ref, xscr):
    f32 = jnp.float32
    bf16 = jnp.bfloat16

    for q in range(_PACK // 2):
        for ci in range(3):
            for par in range(2):
                a = x_ref[base + 2 * q, ci][:, par * 64:(par + 1) * 64]
                b = x_ref[base + 2 * q + 1, ci][:, par * 64:(par + 1) * 64]
                r0 = ci * 64 + par * 32
                xscr[r0:r0 + 32, 128 * q:128 * (q + 1)] = (
                    jnp.concatenate([a, b], axis=1))
    x = xscr[:, :]

    xr, xl = _shift_lr(x, 64)
    xs = jnp.concatenate([xr, x, xl], axis=0).astype(bf16)

    y = jnp.dot(m1_ref[...], xs, preferred_element_type=f32)
    p1 = _pool2x2(y, b1_ref, s1_ref)

    p1r, p1l = _shift_lr(p1, 32)
    ps = jnp.concatenate([p1r, p1, p1l], axis=0).astype(bf16)

    y2 = jnp.dot(m2_ref[...], ps, preferred_element_type=f32)
    p2 = _pool2x2(y2, b2_ref, s2_ref)

    out_ref[0, base * 16:(base + _PACK) * 16, :] = (
        jnp.transpose(p2).astype(out_ref.dtype))


def _conv_stack_kernel(x_ref, m1_ref, b1_ref, m2_ref, b2_ref,
                       s1_ref, s2_ref, out_ref, *xscrs):
    for u in range(_CHAINS):
        _conv_chain(x_ref, u * _PACK, m1_ref, b1_ref, m2_ref, b2_ref,
                    s1_ref, s2_ref, out_ref, xscrs[u])


def _fc_stack_kernel(x_ref, w1_ref, b1_ref, w2_ref, b2_ref, w3_ref, b3_ref,
                     o_ref):
    f32 = jnp.float32
    h = jnp.dot(x_ref[...], w1_ref[...], preferred_element_type=f32)
    h = jnp.maximum(h + b1_ref[...], 0.0)
    h = jnp.dot(h, w2_ref[...], preferred_element_type=f32)
    h = jnp.maximum(h + b2_ref[...], 0.0)
    o = jnp.dot(h, w3_ref[...], preferred_element_type=f32) + b3_ref[...]
    o_ref[...] = o.astype(o_ref.dtype)


def kernel(x, m1_0, m1_1, m1_2, c1_0, c1_2, b1s, re1, ro1, pe1, po1,
           m2_0, m2_1, m2_2, c2_0, c2_2, b2s, re2, ro2, pe2, po2,
           fc1_w, fc1_b, fc2_w, fc2_b, fc3_w, fc3_b):
    f32 = jnp.float32
    bf16 = jnp.bfloat16

    N = x.shape[0]
    assert x.shape[1:] == (3, 64, 64), x.shape
    G = _PACK * _CHAINS
    Np = _round_up(N, G)
    x = x.astype(f32)
    if Np != N:
        x = jnp.pad(x, ((0, Np - N), (0, 0), (0, 0), (0, 0)))
    Nb = Np // G

    ci_ = np.arange(3)[:, None, None]
    par_ = np.arange(2)[None, :, None]
    i2_ = np.arange(32)[None, None, :]
    inperm = (ci_ * 64 + 2 * i2_ + par_).reshape(-1)
    cperm = np.concatenate([t * 192 + inperm for t in range(3)])
    m1 = jnp.concatenate([m1_0, m1_1, m1_2], axis=1)
    m1 = m1[jnp.asarray(_row_par_perm(6, 32)), :]
    m1 = m1[:, jnp.asarray(cperm)].astype(bf16)
    m2 = jnp.concatenate([m2_0, m2_1, m2_2], axis=1)
    m2 = m2[jnp.asarray(_row_par_perm(12, 16)), :].astype(bf16)
    s1 = jnp.asarray(_col_compact_sel(64, _PACK), bf16)
    s2 = jnp.asarray(_col_compact_sel(32, _PACK), bf16)
    b1p = b1s.astype(f32)[::2]
    b2p = b2s.astype(f32)[::2]

    xf = x.reshape(Np, 3, 32, 128)

    conv_out = pl.pallas_call(
        _conv_stack_kernel,
        out_shape=jax.ShapeDtypeStruct((Nb, G * 16, 12 * 16), bf16),
        grid=(Nb,),
        in_specs=[
            pl.BlockSpec((G, 3, 32, 128), lambda i: (i, 0, 0, 0)),
            pl.BlockSpec(m1.shape, lambda i: (0, 0)),
            pl.BlockSpec(b1p.shape, lambda i: (0, 0)),
            pl.BlockSpec(m2.shape, lambda i: (0, 0)),
            pl.BlockSpec(b2p.shape, lambda i: (0, 0)),
            pl.BlockSpec(s1.shape, lambda i: (0, 0)),
            pl.BlockSpec(s2.shape, lambda i: (0, 0)),
        ],
        out_specs=pl.BlockSpec((1, G * 16, 12 * 16), lambda i: (i, 0, 0)),
        scratch_shapes=[pltpu.VMEM((3 * 64, _PACK * 64), f32)
                        for _ in range(_CHAINS)],
        compiler_params=pltpu.CompilerParams(dimension_semantics=("parallel",)),
    )(xf, m1, b1p, m2, b2p, s1, s2)

    flat = conv_out.reshape(Np, 16 * 192)[:N]
    w1p = fc1_w[jnp.asarray(_fc1_perm()), :].astype(bf16)

    K = flat.shape[1]
    n1 = fc1_w.shape[1]
    n2 = fc2_w.shape[1]
    n3 = fc3_w.shape[1]

    TB = min(128, _round_up(N, 8))
    Nf = _round_up(N, TB)
    if Nf != N:
        flat = jnp.pad(flat, ((0, Nf - N), (0, 0)))

    out = pl.pallas_call(
        _fc_stack_kernel,
        out_shape=jax.ShapeDtypeStruct((Nf, n3), f32),
        grid=(Nf // TB,),
        in_specs=[
            pl.BlockSpec((TB, K), lambda i: (i, 0)),
            pl.BlockSpec((K, n1), lambda i: (0, 0)),
            pl.BlockSpec((1, n1), lambda i: (0, 0)),
            pl.BlockSpec((n1, n2), lambda i: (0, 0)),
            pl.BlockSpec((1, n2), lambda i: (0, 0)),
            pl.BlockSpec((n2, n3), lambda i: (0, 0)),
            pl.BlockSpec((1, n3), lambda i: (0, 0)),
        ],
        out_specs=pl.BlockSpec((TB, n3), lambda i: (i, 0)),
        compiler_params=pltpu.CompilerParams(dimension_semantics=("parallel",)),
    )(flat, w1p, fc1_b.astype(f32),
      fc2_w.astype(f32), fc2_b.astype(f32),
      fc3_w.astype(f32), fc3_b.astype(f32))
    return out[:N]


# speedup vs baseline: 2.4199x; 1.0626x over previous
import numpy as np
import jax
import jax.numpy as jnp
from jax.experimental import pallas as pl
from jax.experimental.pallas import tpu as pltpu

_PACK = 8
_CHAINS = 4
_FPAD = 256


def _round_up(n, m):
    return ((n + m - 1) // m) * m


def _col_compact_sel(w, pack):
    S = np.zeros((w, w // 2), np.float32)
    S[2 * np.arange(w // 2), np.arange(w // 2)] = 1.0
    return np.kron(np.eye(pack, dtype=np.float32), S)


def _fc1_perm():
    j = np.arange(16)[:, None, None]
    c = np.arange(12)[None, :, None]
    i = np.arange(16)[None, None, :]
    return (c * 256 + i * 16 + j).reshape(-1)


def _shift_lr(x, img_w):
    R, L = x.shape
    z = jnp.zeros((R, 1), x.dtype)
    xl = jnp.concatenate([x[:, 1:], z], axis=1)
    xr = jnp.concatenate([z, x[:, :-1]], axis=1)
    col = jax.lax.broadcasted_iota(jnp.int32, (1, L), 1) % img_w
    xl = jnp.where(col == img_w - 1, jnp.zeros((), x.dtype), xl)
    xr = jnp.where(col == 0, jnp.zeros((), x.dtype), xr)
    return xr, xl


def _pool2x2(y, b_ref, sel_ref):
    R, L = y.shape
    tr = jnp.maximum(y[:R // 2, :], y[R // 2:, :]) + b_ref[...]
    u = jnp.maximum(tr, jnp.concatenate([tr[:, 1:], tr[:, :1]], axis=1))
    p = jnp.dot(u.astype(sel_ref.dtype), sel_ref[:L, :],
                preferred_element_type=jnp.float32)
    return jnp.maximum(p, 0.0)


def _row_par_perm(C, Ho):
    par = np.arange(2)[:, None, None]
    c = np.arange(C)[None, :, None]
    i2 = np.arange(Ho)[None, None, :]
    return (c * 2 * Ho + 2 * i2 + par).reshape(-1)


def _conv_chain(x_ref, base, m1_ref, b1_ref, m2_ref, b2_ref,
                s1_ref, s2_ref, out_ref, xscr):
    f32 = jnp.float32
    bf16 = jnp.bfloat16

    for q in range(_PACK // 2):
        for ci in range(3):
            for par in range(2):
                a = x_ref[base + 2 * q, ci][:, par * 64:(par + 1) * 64]
                b = x_ref[base + 2 * q + 1, ci][:, par * 64:(par + 1) * 64]
                r0 = ci * 64 + par * 32
                xscr[r0:r0 + 32, 128 * q:128 * (q + 1)] = (
                    jnp.concatenate([a, b], axis=1))
    x = xscr[:, :]

    xr, xl = _shift_lr(x, 64)
    xs = jnp.concatenate([xr, x, xl], axis=0).astype(bf16)

    y = jnp.dot(m1_ref[...], xs, preferred_element_type=f32)
    p1 = _pool2x2(y, b1_ref, s1_ref)

    p1r, p1l = _shift_lr(p1, 32)
    ps = jnp.concatenate([p1r, p1, p1l], axis=0).astype(bf16)

    y2 = jnp.dot(m2_ref[...], ps, preferred_element_type=f32)
    p2 = _pool2x2(y2, b2_ref, s2_ref)

    out_ref[0, base * 16:(base + _PACK) * 16, :192] = (
        jnp.transpose(p2).astype(out_ref.dtype))


def _conv_stack_kernel(x_ref, m1_ref, b1_ref, m2_ref, b2_ref,
                       s1_ref, s2_ref, out_ref, *xscrs):
    out_ref[0, :, 192:] = jnp.zeros(
        (_CHAINS * _PACK * 16, _FPAD - 192), out_ref.dtype)
    for u in range(_CHAINS):
        _conv_chain(x_ref, u * _PACK, m1_ref, b1_ref, m2_ref, b2_ref,
                    s1_ref, s2_ref, out_ref, xscrs[u])


def _fc_stack_kernel(x_ref, w1_ref, b1_ref, w2_ref, b2_ref, w3_ref, b3_ref,
                     o_ref):
    f32 = jnp.float32
    h = jnp.dot(x_ref[...], w1_ref[...], preferred_element_type=f32)
    h = jnp.maximum(h + b1_ref[...], 0.0)
    h = jnp.dot(h, w2_ref[...], preferred_element_type=f32)
    h = jnp.maximum(h + b2_ref[...], 0.0)
    o = jnp.dot(h, w3_ref[...], preferred_element_type=f32) + b3_ref[...]
    o_ref[...] = o.astype(o_ref.dtype)


def kernel(x, m1_0, m1_1, m1_2, c1_0, c1_2, b1s, re1, ro1, pe1, po1,
           m2_0, m2_1, m2_2, c2_0, c2_2, b2s, re2, ro2, pe2, po2,
           fc1_w, fc1_b, fc2_w, fc2_b, fc3_w, fc3_b):
    f32 = jnp.float32
    bf16 = jnp.bfloat16

    N = x.shape[0]
    assert x.shape[1:] == (3, 64, 64), x.shape
    G = _PACK * _CHAINS
    Np = _round_up(N, G)
    x = x.astype(f32)
    if Np != N:
        x = jnp.pad(x, ((0, Np - N), (0, 0), (0, 0), (0, 0)))
    Nb = Np // G

    ci_ = np.arange(3)[:, None, None]
    par_ = np.arange(2)[None, :, None]
    i2_ = np.arange(32)[None, None, :]
    inperm = (ci_ * 64 + 2 * i2_ + par_).reshape(-1)
    cperm = np.concatenate([t * 192 + inperm for t in range(3)])
    m1 = jnp.concatenate([m1_0, m1_1, m1_2], axis=1)
    m1 = m1[jnp.asarray(_row_par_perm(6, 32)), :]
    m1 = m1[:, jnp.asarray(cperm)].astype(bf16)
    m2 = jnp.concatenate([m2_0, m2_1, m2_2], axis=1)
    m2 = m2[jnp.asarray(_row_par_perm(12, 16)), :].astype(bf16)
    s1 = jnp.asarray(_col_compact_sel(64, _PACK), bf16)
    s2 = jnp.asarray(_col_compact_sel(32, _PACK), bf16)
    b1p = b1s.astype(f32)[::2]
    b2p = b2s.astype(f32)[::2]

    xf = x.reshape(Np, 3, 32, 128)

    conv_out = pl.pallas_call(
        _conv_stack_kernel,
        out_shape=jax.ShapeDtypeStruct((Nb, G * 16, _FPAD), bf16),
        grid=(Nb,),
        in_specs=[
            pl.BlockSpec((G, 3, 32, 128), lambda i: (i, 0, 0, 0)),
            pl.BlockSpec(m1.shape, lambda i: (0, 0)),
            pl.BlockSpec(b1p.shape, lambda i: (0, 0)),
            pl.BlockSpec(m2.shape, lambda i: (0, 0)),
            pl.BlockSpec(b2p.shape, lambda i: (0, 0)),
            pl.BlockSpec(s1.shape, lambda i: (0, 0)),
            pl.BlockSpec(s2.shape, lambda i: (0, 0)),
        ],
        out_specs=pl.BlockSpec((1, G * 16, _FPAD), lambda i: (i, 0, 0)),
        scratch_shapes=[pltpu.VMEM((3 * 64, _PACK * 64), f32)
                        for _ in range(_CHAINS)],
        compiler_params=pltpu.CompilerParams(dimension_semantics=("parallel",)),
    )(xf, m1, b1p, m2, b2p, s1, s2)

    flat = conv_out.reshape(Np, 16 * _FPAD)[:N]
    w1p = fc1_w[jnp.asarray(_fc1_perm()), :].astype(bf16)
    w1p = jnp.pad(w1p.reshape(16, 192, -1),
                  ((0, 0), (0, _FPAD - 192), (0, 0))).reshape(16 * _FPAD, -1)

    K = 16 * _FPAD
    n1 = fc1_w.shape[1]
    n2 = fc2_w.shape[1]
    n3 = fc3_w.shape[1]

    TB = min(128, _round_up(N, 8))
    Nf = _round_up(N, TB)
    if Nf != N:
        flat = jnp.pad(flat, ((0, Nf - N), (0, 0)))

    out = pl.pallas_call(
        _fc_stack_kernel,
        out_shape=jax.ShapeDtypeStruct((Nf, n3), f32),
        grid=(Nf // TB,),
        in_specs=[
            pl.BlockSpec((TB, K), lambda i: (i, 0)),
            pl.BlockSpec((K, n1), lambda i: (0, 0)),
            pl.BlockSpec((1, n1), lambda i: (0, 0)),
            pl.BlockSpec((n1, n2), lambda i: (0, 0)),
            pl.BlockSpec((1, n2), lambda i: (0, 0)),
            pl.BlockSpec((n2, n3), lambda i: (0, 0)),
            pl.BlockSpec((1, n3), lambda i: (0, 0)),
        ],
        out_specs=pl.BlockSpec((TB, n3), lambda i: (i, 0)),
        compiler_params=pltpu.CompilerParams(dimension_semantics=("parallel",)),
    )(flat, w1p, fc1_b.astype(f32),
      fc2_w.astype(f32), fc2_b.astype(f32),
      fc3_w.astype(f32), fc3_b.astype(f32))
    return out[:N]


# speedup vs baseline: 2.4752x; 1.0229x over previous
import numpy as np
import jax
import jax.numpy as jnp
from jax.experimental import pallas as pl
from jax.experimental.pallas import tpu as pltpu

_PACK = 8
_CHAINS = 8
_FPAD = 256


def _round_up(n, m):
    return ((n + m - 1) // m) * m


def _col_compact_sel(w, pack):
    S = np.zeros((w, w // 2), np.float32)
    S[2 * np.arange(w // 2), np.arange(w // 2)] = 1.0
    return np.kron(np.eye(pack, dtype=np.float32), S)


def _fc1_perm():
    j = np.arange(16)[:, None, None]
    c = np.arange(12)[None, :, None]
    i = np.arange(16)[None, None, :]
    return (c * 256 + i * 16 + j).reshape(-1)


def _shift_lr(x, img_w):
    R, L = x.shape
    z = jnp.zeros((R, 1), x.dtype)
    xl = jnp.concatenate([x[:, 1:], z], axis=1)
    xr = jnp.concatenate([z, x[:, :-1]], axis=1)
    col = jax.lax.broadcasted_iota(jnp.int32, (1, L), 1) % img_w
    xl = jnp.where(col == img_w - 1, jnp.zeros((), x.dtype), xl)
    xr = jnp.where(col == 0, jnp.zeros((), x.dtype), xr)
    return xr, xl


def _pool2x2(y, b_ref, sel_ref):
    R, L = y.shape
    tr = jnp.maximum(y[:R // 2, :], y[R // 2:, :]) + b_ref[...]
    u = jnp.maximum(tr, jnp.concatenate([tr[:, 1:], tr[:, :1]], axis=1))
    p = jnp.dot(u.astype(sel_ref.dtype), sel_ref[:L, :],
                preferred_element_type=jnp.float32)
    return jnp.maximum(p, 0.0)


def _row_par_perm(C, Ho):
    par = np.arange(2)[:, None, None]
    c = np.arange(C)[None, :, None]
    i2 = np.arange(Ho)[None, None, :]
    return (c * 2 * Ho + 2 * i2 + par).reshape(-1)


def _conv_chain(x_ref, base, m1_ref, b1_ref, m2_ref, b2_ref,
                s1_ref, s2_ref, out_ref, xscr):
    f32 = jnp.float32
    bf16 = jnp.bfloat16

    for q in range(_PACK // 2):
        for ci in range(3):
            for par in range(2):
                a = x_ref[base + 2 * q, ci][:, par * 64:(par + 1) * 64]
                b = x_ref[base + 2 * q + 1, ci][:, par * 64:(par + 1) * 64]
                r0 = ci * 64 + par * 32
                xscr[r0:r0 + 32, 128 * q:128 * (q + 1)] = (
                    jnp.concatenate([a, b], axis=1))
    x = xscr[:, :]

    xr, xl = _shift_lr(x, 64)
    xs = jnp.concatenate([xr, x, xl], axis=0).astype(bf16)

    y = jnp.dot(m1_ref[...], xs, preferred_element_type=f32)
    p1 = _pool2x2(y, b1_ref, s1_ref)

    p1r, p1l = _shift_lr(p1, 32)
    ps = jnp.concatenate([p1r, p1, p1l], axis=0).astype(bf16)

    y2 = jnp.dot(m2_ref[...], ps, preferred_element_type=f32)
    p2 = _pool2x2(y2, b2_ref, s2_ref)

    out_ref[0, base * 16:(base + _PACK) * 16, :192] = (
        jnp.transpose(p2).astype(out_ref.dtype))


def _conv_stack_kernel(x_ref, m1_ref, b1_ref, m2_ref, b2_ref,
                       s1_ref, s2_ref, out_ref, *xscrs):
    out_ref[0, :, 192:] = jnp.zeros(
        (_CHAINS * _PACK * 16, _FPAD - 192), out_ref.dtype)
    for u in range(_CHAINS):
        _conv_chain(x_ref, u * _PACK, m1_ref, b1_ref, m2_ref, b2_ref,
                    s1_ref, s2_ref, out_ref, xscrs[u])


def _fc_stack_kernel(x_ref, w1_ref, b1_ref, w2_ref, b2_ref, w3_ref, b3_ref,
                     o_ref):
    f32 = jnp.float32
    h = jnp.dot(x_ref[...], w1_ref[...], preferred_element_type=f32)
    h = jnp.maximum(h + b1_ref[...], 0.0)
    h = jnp.dot(h, w2_ref[...], preferred_element_type=f32)
    h = jnp.maximum(h + b2_ref[...], 0.0)
    o = jnp.dot(h, w3_ref[...], preferred_element_type=f32) + b3_ref[...]
    o_ref[...] = o.astype(o_ref.dtype)


def kernel(x, m1_0, m1_1, m1_2, c1_0, c1_2, b1s, re1, ro1, pe1, po1,
           m2_0, m2_1, m2_2, c2_0, c2_2, b2s, re2, ro2, pe2, po2,
           fc1_w, fc1_b, fc2_w, fc2_b, fc3_w, fc3_b):
    f32 = jnp.float32
    bf16 = jnp.bfloat16

    N = x.shape[0]
    assert x.shape[1:] == (3, 64, 64), x.shape
    G = _PACK * _CHAINS
    Np = _round_up(N, G)
    x = x.astype(f32)
    if Np != N:
        x = jnp.pad(x, ((0, Np - N), (0, 0), (0, 0), (0, 0)))
    Nb = Np // G

    ci_ = np.arange(3)[:, None, None]
    par_ = np.arange(2)[None, :, None]
    i2_ = np.arange(32)[None, None, :]
    inperm = (ci_ * 64 + 2 * i2_ + par_).reshape(-1)
    cperm = np.concatenate([t * 192 + inperm for t in range(3)])
    m1 = jnp.concatenate([m1_0, m1_1, m1_2], axis=1)
    m1 = m1[jnp.asarray(_row_par_perm(6, 32)), :]
    m1 = m1[:, jnp.asarray(cperm)].astype(bf16)
    m2 = jnp.concatenate([m2_0, m2_1, m2_2], axis=1)
    m2 = m2[jnp.asarray(_row_par_perm(12, 16)), :].astype(bf16)
    s1 = jnp.asarray(_col_compact_sel(64, _PACK), bf16)
    s2 = jnp.asarray(_col_compact_sel(32, _PACK), bf16)
    b1p = b1s.astype(f32)[::2]
    b2p = b2s.astype(f32)[::2]

    xf = x.reshape(Np, 3, 32, 128)

    conv_out = pl.pallas_call(
        _conv_stack_kernel,
        out_shape=jax.ShapeDtypeStruct((Nb, G * 16, _FPAD), bf16),
        grid=(Nb,),
        in_specs=[
            pl.BlockSpec((G, 3, 32, 128), lambda i: (i, 0, 0, 0)),
            pl.BlockSpec(m1.shape, lambda i: (0, 0)),
            pl.BlockSpec(b1p.shape, lambda i: (0, 0)),
            pl.BlockSpec(m2.shape, lambda i: (0, 0)),
            pl.BlockSpec(b2p.shape, lambda i: (0, 0)),
            pl.BlockSpec(s1.shape, lambda i: (0, 0)),
            pl.BlockSpec(s2.shape, lambda i: (0, 0)),
        ],
        out_specs=pl.BlockSpec((1, G * 16, _FPAD), lambda i: (i, 0, 0)),
        scratch_shapes=[pltpu.VMEM((3 * 64, _PACK * 64), f32)
                        for _ in range(_CHAINS)],
        compiler_params=pltpu.CompilerParams(dimension_semantics=("parallel",)),
    )(xf, m1, b1p, m2, b2p, s1, s2)

    flat = conv_out.reshape(Np, 16 * _FPAD)[:N]
    w1p = fc1_w[jnp.asarray(_fc1_perm()), :].astype(bf16)
    w1p = jnp.pad(w1p.reshape(16, 192, -1),
                  ((0, 0), (0, _FPAD - 192), (0, 0))).reshape(16 * _FPAD, -1)

    K = 16 * _FPAD
    n1 = fc1_w.shape[1]
    n2 = fc2_w.shape[1]
    n3 = fc3_w.shape[1]

    TB = min(128, _round_up(N, 8))
    Nf = _round_up(N, TB)
    if Nf != N:
        flat = jnp.pad(flat, ((0, Nf - N), (0, 0)))

    out = pl.pallas_call(
        _fc_stack_kernel,
        out_shape=jax.ShapeDtypeStruct((Nf, n3), f32),
        grid=(Nf // TB,),
        in_specs=[
            pl.BlockSpec((TB, K), lambda i: (i, 0)),
            pl.BlockSpec((K, n1), lambda i: (0, 0)),
            pl.BlockSpec((1, n1), lambda i: (0, 0)),
            pl.BlockSpec((n1, n2), lambda i: (0, 0)),
            pl.BlockSpec((1, n2), lambda i: (0, 0)),
            pl.BlockSpec((n2, n3), lambda i: (0, 0)),
            pl.BlockSpec((1, n3), lambda i: (0, 0)),
        ],
        out_specs=pl.BlockSpec((TB, n3), lambda i: (i, 0)),
        compiler_params=pltpu.CompilerParams(dimension_semantics=("parallel",)),
    )(flat, w1p, fc1_b.astype(f32),
      fc2_w.astype(f32), fc2_b.astype(f32),
      fc3_w.astype(f32), fc3_b.astype(f32))
    return out[:N]


# speedup vs baseline: 2.4780x; 1.0011x over previous
import numpy as np
import jax
import jax.numpy as jnp
from jax.experimental import pallas as pl
from jax.experimental.pallas import tpu as pltpu

_PACK = 8
_CHAINS = 8
_FPAD = 256


def _round_up(n, m):
    return ((n + m - 1) // m) * m


def _col_compact_sel(w, pack):
    S = np.zeros((w, w // 2), np.float32)
    S[2 * np.arange(w // 2), np.arange(w // 2)] = 1.0
    return np.kron(np.eye(pack, dtype=np.float32), S)


def _fc1_perm():
    j = np.arange(16)[:, None, None]
    c = np.arange(12)[None, :, None]
    i = np.arange(16)[None, None, :]
    return (c * 256 + i * 16 + j).reshape(-1)


def _shift_lr(x, img_w):
    R, L = x.shape
    z = jnp.zeros((R, 1), x.dtype)
    xl = jnp.concatenate([x[:, 1:], z], axis=1)
    xr = jnp.concatenate([z, x[:, :-1]], axis=1)
    col = jax.lax.broadcasted_iota(jnp.int32, (1, L), 1) % img_w
    xl = jnp.where(col == img_w - 1, jnp.zeros((), x.dtype), xl)
    xr = jnp.where(col == 0, jnp.zeros((), x.dtype), xr)
    return xr, xl


def _pool2x2(y, b_ref, sel_ref):
    R, L = y.shape
    tr = jnp.maximum(y[:R // 2, :], y[R // 2:, :]) + b_ref[...]
    u = jnp.maximum(tr, jnp.concatenate([tr[:, 1:], tr[:, :1]], axis=1))
    p = jnp.dot(u.astype(sel_ref.dtype), sel_ref[:L, :],
                preferred_element_type=jnp.float32)
    return jnp.maximum(p, 0.0)


def _row_par_perm(C, Ho):
    par = np.arange(2)[:, None, None]
    c = np.arange(C)[None, :, None]
    i2 = np.arange(Ho)[None, None, :]
    return (c * 2 * Ho + 2 * i2 + par).reshape(-1)


def _conv_chain(x_ref, base, m1_ref, b1_ref, m2_ref, b2_ref,
                s1_ref, s2_ref, out_ref):
    f32 = jnp.float32
    bf16 = jnp.bfloat16

    rows = []
    for ci in range(3):
        for par in range(2):
            rows.append(jnp.concatenate(
                [x_ref[base + pk, ci][:, par * 64:(par + 1) * 64]
                 for pk in range(_PACK)], axis=1))
    x = jnp.concatenate(rows, axis=0)

    xr, xl = _shift_lr(x, 64)
    xs = jnp.concatenate([xr, x, xl], axis=0).astype(bf16)

    y = jnp.dot(m1_ref[...], xs, preferred_element_type=f32)
    p1 = _pool2x2(y, b1_ref, s1_ref)

    p1r, p1l = _shift_lr(p1, 32)
    ps = jnp.concatenate([p1r, p1, p1l], axis=0).astype(bf16)

    y2 = jnp.dot(m2_ref[...], ps, preferred_element_type=f32)
    p2 = _pool2x2(y2, b2_ref, s2_ref)

    out_ref[0, base * 16:(base + _PACK) * 16, :192] = (
        jnp.transpose(p2).astype(out_ref.dtype))


def _conv_stack_kernel(x_ref, m1_ref, b1_ref, m2_ref, b2_ref,
                       s1_ref, s2_ref, out_ref):
    out_ref[0, :, 192:] = jnp.zeros(
        (_CHAINS * _PACK * 16, _FPAD - 192), out_ref.dtype)
    for u in range(_CHAINS):
        _conv_chain(x_ref, u * _PACK, m1_ref, b1_ref, m2_ref, b2_ref,
                    s1_ref, s2_ref, out_ref)


def _fc_stack_kernel(x_ref, w1_ref, b1_ref, w2_ref, b2_ref, w3_ref, b3_ref,
                     o_ref):
    f32 = jnp.float32
    h = jnp.dot(x_ref[...], w1_ref[...], preferred_element_type=f32)
    h = jnp.maximum(h + b1_ref[...], 0.0)
    h = jnp.dot(h, w2_ref[...], preferred_element_type=f32)
    h = jnp.maximum(h + b2_ref[...], 0.0)
    o = jnp.dot(h, w3_ref[...], preferred_element_type=f32) + b3_ref[...]
    o_ref[...] = o.astype(o_ref.dtype)


def kernel(x, m1_0, m1_1, m1_2, c1_0, c1_2, b1s, re1, ro1, pe1, po1,
           m2_0, m2_1, m2_2, c2_0, c2_2, b2s, re2, ro2, pe2, po2,
           fc1_w, fc1_b, fc2_w, fc2_b, fc3_w, fc3_b):
    f32 = jnp.float32
    bf16 = jnp.bfloat16

    N = x.shape[0]
    assert x.shape[1:] == (3, 64, 64), x.shape
    G = _PACK * _CHAINS
    Np = _round_up(N, G)
    x = x.astype(f32)
    if Np != N:
        x = jnp.pad(x, ((0, Np - N), (0, 0), (0, 0), (0, 0)))
    Nb = Np // G

    ci_ = np.arange(3)[:, None, None]
    par_ = np.arange(2)[None, :, None]
    i2_ = np.arange(32)[None, None, :]
    inperm = (ci_ * 64 + 2 * i2_ + par_).reshape(-1)
    cperm = np.concatenate([t * 192 + inperm for t in range(3)])
    m1 = jnp.concatenate([m1_0, m1_1, m1_2], axis=1)
    m1 = m1[jnp.asarray(_row_par_perm(6, 32)), :]
    m1 = m1[:, jnp.asarray(cperm)].astype(bf16)
    m2 = jnp.concatenate([m2_0, m2_1, m2_2], axis=1)
    m2 = m2[jnp.asarray(_row_par_perm(12, 16)), :].astype(bf16)
    s1 = jnp.asarray(_col_compact_sel(64, _PACK), bf16)
    s2 = jnp.asarray(_col_compact_sel(32, _PACK), bf16)
    b1p = b1s.astype(f32)[::2]
    b2p = b2s.astype(f32)[::2]

    xf = x.reshape(Np, 3, 32, 128)

    conv_out = pl.pallas_call(
        _conv_stack_kernel,
        out_shape=jax.ShapeDtypeStruct((Nb, G * 16, _FPAD), bf16),
        grid=(Nb,),
        in_specs=[
            pl.BlockSpec((G, 3, 32, 128), lambda i: (i, 0, 0, 0)),
            pl.BlockSpec(m1.shape, lambda i: (0, 0)),
            pl.BlockSpec(b1p.shape, lambda i: (0, 0)),
            pl.BlockSpec(m2.shape, lambda i: (0, 0)),
            pl.BlockSpec(b2p.shape, lambda i: (0, 0)),
            pl.BlockSpec(s1.shape, lambda i: (0, 0)),
            pl.BlockSpec(s2.shape, lambda i: (0, 0)),
        ],
        out_specs=pl.BlockSpec((1, G * 16, _FPAD), lambda i: (i, 0, 0)),
        compiler_params=pltpu.CompilerParams(dimension_semantics=("parallel",)),
    )(xf, m1, b1p, m2, b2p, s1, s2)

    flat = conv_out.reshape(Np, 16 * _FPAD)[:N]
    w1p = fc1_w[jnp.asarray(_fc1_perm()), :].astype(bf16)
    w1p = jnp.pad(w1p.reshape(16, 192, -1),
                  ((0, 0), (0, _FPAD - 192), (0, 0))).reshape(16 * _FPAD, -1)

    K = 16 * _FPAD
    n1 = fc1_w.shape[1]
    n2 = fc2_w.shape[1]
    n3 = fc3_w.shape[1]

    TB = min(128, _round_up(N, 8))
    Nf = _round_up(N, TB)
    if Nf != N:
        flat = jnp.pad(flat, ((0, Nf - N), (0, 0)))

    out = pl.pallas_call(
        _fc_stack_kernel,
        out_shape=jax.ShapeDtypeStruct((Nf, n3), f32),
        grid=(Nf // TB,),
        in_specs=[
            pl.BlockSpec((TB, K), lambda i: (i, 0)),
            pl.BlockSpec((K, n1), lambda i: (0, 0)),
            pl.BlockSpec((1, n1), lambda i: (0, 0)),
            pl.BlockSpec((n1, n2), lambda i: (0, 0)),
            pl.BlockSpec((1, n2), lambda i: (0, 0)),
            pl.BlockSpec((n2, n3), lambda i: (0, 0)),
            pl.BlockSpec((1, n3), lambda i: (0, 0)),
        ],
        out_specs=pl.BlockSpec((TB, n3), lambda i: (i, 0)),
        compiler_params=pltpu.CompilerParams(dimension_semantics=("parallel",)),
    )(flat, w1p, fc1_b.astype(f32),
      fc2_w.astype(f32), fc2_b.astype(f32),
      fc3_w.astype(f32), fc3_b.astype(f32))
    return out[:N]


# speedup vs baseline: 2.4863x; 1.0034x over previous
import numpy as np
import jax
import jax.numpy as jnp
from jax.experimental import pallas as pl
from jax.experimental.pallas import tpu as pltpu

_PACK = 8
_CHAINS = 8
_FPAD = 256


def _round_up(n, m):
    return ((n + m - 1) // m) * m


def _col_compact_sel(w, pack):
    S = np.zeros((w, w // 2), np.float32)
    S[2 * np.arange(w // 2), np.arange(w // 2)] = 1.0
    return np.kron(np.eye(pack, dtype=np.float32), S)


def _fc1_perm():
    j = np.arange(16)[:, None, None]
    c = np.arange(12)[None, :, None]
    i = np.arange(16)[None, None, :]
    return (c * 256 + i * 16 + j).reshape(-1)


def _shift_lr(x, img_w):
    R, L = x.shape
    z = jnp.zeros((R, 1), x.dtype)
    xl = jnp.concatenate([x[:, 1:], z], axis=1)
    xr = jnp.concatenate([z, x[:, :-1]], axis=1)
    col = jax.lax.broadcasted_iota(jnp.int32, (1, L), 1) % img_w
    xl = jnp.where(col == img_w - 1, jnp.zeros((), x.dtype), xl)
    xr = jnp.where(col == 0, jnp.zeros((), x.dtype), xr)
    return xr, xl


def _pool2x2(y, b_ref, sel_ref):
    R, L = y.shape
    tr = jnp.maximum(y[:R // 2, :], y[R // 2:, :]) + b_ref[...]
    u = jnp.maximum(tr, jnp.concatenate([tr[:, 1:], tr[:, :1]], axis=1))
    p = jnp.dot(u.astype(sel_ref.dtype), sel_ref[:L, :],
                preferred_element_type=jnp.float32)
    return jnp.maximum(p, 0.0)


def _row_par_perm(C, Ho):
    par = np.arange(2)[:, None, None]
    c = np.arange(C)[None, :, None]
    i2 = np.arange(Ho)[None, None, :]
    return (c * 2 * Ho + 2 * i2 + par).reshape(-1)


def _conv_chain(x_ref, base, m1_ref, b1_ref, m2_ref, b2_ref,
                s1_ref, s2_ref, out_ref):
    f32 = jnp.float32
    bf16 = jnp.bfloat16

    rows = []
    for ci in range(3):
        for par in range(2):
            rows.append(jnp.concatenate(
                [x_ref[base + pk, ci][:, par * 64:(par + 1) * 64]
                 for pk in range(_PACK)], axis=1))
    x = jnp.concatenate(rows, axis=0)

    xb = x.astype(bf16)
    xr, xl = _shift_lr(xb, 64)
    xs = jnp.concatenate([xr, xb, xl], axis=0)

    y = jnp.dot(m1_ref[...], xs, preferred_element_type=f32)
    p1 = _pool2x2(y, b1_ref, s1_ref)

    p1b = p1.astype(bf16)
    p1r, p1l = _shift_lr(p1b, 32)
    ps = jnp.concatenate([p1r, p1b, p1l], axis=0)

    y2 = jnp.dot(m2_ref[...], ps, preferred_element_type=f32)
    p2 = _pool2x2(y2, b2_ref, s2_ref)

    out_ref[0, base * 16:(base + _PACK) * 16, :192] = (
        jnp.transpose(p2).astype(out_ref.dtype))


def _conv_stack_kernel(x_ref, m1_ref, b1_ref, m2_ref, b2_ref,
                       s1_ref, s2_ref, out_ref):
    out_ref[0, :, 192:] = jnp.zeros(
        (_CHAINS * _PACK * 16, _FPAD - 192), out_ref.dtype)
    for u in range(_CHAINS):
        _conv_chain(x_ref, u * _PACK, m1_ref, b1_ref, m2_ref, b2_ref,
                    s1_ref, s2_ref, out_ref)


def _fc_stack_kernel(x_ref, w1_ref, b1_ref, w2_ref, b2_ref, w3_ref, b3_ref,
                     o_ref):
    f32 = jnp.float32
    h = jnp.dot(x_ref[...], w1_ref[...], preferred_element_type=f32)
    h = jnp.maximum(h + b1_ref[...], 0.0)
    h = jnp.dot(h, w2_ref[...], preferred_element_type=f32)
    h = jnp.maximum(h + b2_ref[...], 0.0)
    o = jnp.dot(h, w3_ref[...], preferred_element_type=f32) + b3_ref[...]
    o_ref[...] = o.astype(o_ref.dtype)


def kernel(x, m1_0, m1_1, m1_2, c1_0, c1_2, b1s, re1, ro1, pe1, po1,
           m2_0, m2_1, m2_2, c2_0, c2_2, b2s, re2, ro2, pe2, po2,
           fc1_w, fc1_b, fc2_w, fc2_b, fc3_w, fc3_b):
    f32 = jnp.float32
    bf16 = jnp.bfloat16

    N = x.shape[0]
    assert x.shape[1:] == (3, 64, 64), x.shape
    G = _PACK * _CHAINS
    Np = _round_up(N, G)
    x = x.astype(f32)
    if Np != N:
        x = jnp.pad(x, ((0, Np - N), (0, 0), (0, 0), (0, 0)))
    Nb = Np // G

    ci_ = np.arange(3)[:, None, None]
    par_ = np.arange(2)[None, :, None]
    i2_ = np.arange(32)[None, None, :]
    inperm = (ci_ * 64 + 2 * i2_ + par_).reshape(-1)
    cperm = np.concatenate([t * 192 + inperm for t in range(3)])
    m1 = jnp.concatenate([m1_0, m1_1, m1_2], axis=1)
    m1 = m1[jnp.asarray(_row_par_perm(6, 32)), :]
    m1 = m1[:, jnp.asarray(cperm)].astype(bf16)
    m2 = jnp.concatenate([m2_0, m2_1, m2_2], axis=1)
    m2 = m2[jnp.asarray(_row_par_perm(12, 16)), :].astype(bf16)
    s1 = jnp.asarray(_col_compact_sel(64, _PACK), bf16)
    s2 = jnp.asarray(_col_compact_sel(32, _PACK), bf16)
    b1p = b1s.astype(f32)[::2]
    b2p = b2s.astype(f32)[::2]

    xf = x.reshape(Np, 3, 32, 128)

    conv_out = pl.pallas_call(
        _conv_stack_kernel,
        out_shape=jax.ShapeDtypeStruct((Nb, G * 16, _FPAD), bf16),
        grid=(Nb,),
        in_specs=[
            pl.BlockSpec((G, 3, 32, 128), lambda i: (i, 0, 0, 0)),
            pl.BlockSpec(m1.shape, lambda i: (0, 0)),
            pl.BlockSpec(b1p.shape, lambda i: (0, 0)),
            pl.BlockSpec(m2.shape, lambda i: (0, 0)),
            pl.BlockSpec(b2p.shape, lambda i: (0, 0)),
            pl.BlockSpec(s1.shape, lambda i: (0, 0)),
            pl.BlockSpec(s2.shape, lambda i: (0, 0)),
        ],
        out_specs=pl.BlockSpec((1, G * 16, _FPAD), lambda i: (i, 0, 0)),
        compiler_params=pltpu.CompilerParams(dimension_semantics=("parallel",)),
    )(xf, m1, b1p, m2, b2p, s1, s2)

    flat = conv_out.reshape(Np, 16 * _FPAD)[:N]
    w1p = fc1_w[jnp.asarray(_fc1_perm()), :].astype(bf16)
    w1p = jnp.pad(w1p.reshape(16, 192, -1),
                  ((0, 0), (0, _FPAD - 192), (0, 0))).reshape(16 * _FPAD, -1)

    K = 16 * _FPAD
    n1 = fc1_w.shape[1]
    n2 = fc2_w.shape[1]
    n3 = fc3_w.shape[1]

    TB = min(128, _round_up(N, 8))
    Nf = _round_up(N, TB)
    if Nf != N:
        flat = jnp.pad(flat, ((0, Nf - N), (0, 0)))

    out = pl.pallas_call(
        _fc_stack_kernel,
        out_shape=jax.ShapeDtypeStruct((Nf, n3), f32),
        grid=(Nf // TB,),
        in_specs=[
            pl.BlockSpec((TB, K), lambda i: (i, 0)),
            pl.BlockSpec((K, n1), lambda i: (0, 0)),
            pl.BlockSpec((1, n1), lambda i: (0, 0)),
            pl.BlockSpec((n1, n2), lambda i: (0, 0)),
            pl.BlockSpec((1, n2), lambda i: (0, 0)),
            pl.BlockSpec((n2, n3), lambda i: (0, 0)),
            pl.BlockSpec((1, n3), lambda i: (0, 0)),
        ],
        out_specs=pl.BlockSpec((TB, n3), lambda i: (i, 0)),
        compiler_params=pltpu.CompilerParams(dimension_semantics=("parallel",)),
    )(flat, w1p, fc1_b.astype(f32),
      fc2_w.astype(f32), fc2_b.astype(f32),
      fc3_w.astype(f32), fc3_b.astype(f32))
    return out[:N]
